# speedup vs baseline: 1.0040x; 1.0040x over previous
import jax
import jax.numpy as jnp
from jax.experimental import pallas as pl
from jax.experimental.pallas import tpu as pltpu


def _body(idx_ref, x_ref, sp_ref, im_ref, o_ref):
    h = sp_ref.shape[-1]
    row = im_ref[0, pl.ds(idx_ref[0], 1), :]
    o_ref[:, :, :h] = x_ref[:, :, :h] + sp_ref[:]
    o_ref[:, :, h:] = x_ref[:, :, h:] + row[None]


def kernel(x, image_idx, spatial_pos_embed, image_pos_embed):
    B, P, E = x.shape
    H = E // 2
    M = image_pos_embed.shape[1]
    idx = jnp.asarray(image_idx, jnp.int32).reshape(1)
    bb = 2
    return pl.pallas_call(
        _body,
        grid=(B // bb,),
        in_specs=[
            pl.BlockSpec(memory_space=pltpu.SMEM),
            pl.BlockSpec((bb, P, E), lambda b: (b, 0, 0)),
            pl.BlockSpec((1, P, H), lambda b: (0, 0, 0)),
            pl.BlockSpec((1, M, H), lambda b: (0, 0, 0)),
        ],
        out_specs=pl.BlockSpec((bb, P, E), lambda b: (b, 0, 0)),
        out_shape=jax.ShapeDtypeStruct((B, P, E), x.dtype),
        compiler_params=pltpu.CompilerParams(
            dimension_semantics=("arbitrary",),
        ),
    )(idx, x, spatial_pos_embed, image_pos_embed)


# speedup vs baseline: 1.0144x; 1.0104x over previous
import jax
import jax.numpy as jnp
from jax.experimental import pallas as pl
from jax.experimental.pallas import tpu as pltpu


def _body(idx_ref, x_ref, sp_ref, im_ref, o_ref):
    h = sp_ref.shape[-1]
    row = im_ref[0, pl.ds(idx_ref[0], 1), :]
    o_ref[:, :, :h] = x_ref[:, :, :h] + sp_ref[:]
    o_ref[:, :, h:] = x_ref[:, :, h:] + row[None]


def kernel(x, image_idx, spatial_pos_embed, image_pos_embed):
    B, P, E = x.shape
    H = E // 2
    M = image_pos_embed.shape[1]
    idx = jnp.asarray(image_idx, jnp.int32).reshape(1)
    bb = 4
    return pl.pallas_call(
        _body,
        grid=(B // bb,),
        in_specs=[
            pl.BlockSpec(memory_space=pltpu.SMEM),
            pl.BlockSpec((bb, P, E), lambda b: (b, 0, 0)),
            pl.BlockSpec((1, P, H), lambda b: (0, 0, 0)),
            pl.BlockSpec((1, M, H), lambda b: (0, 0, 0)),
        ],
        out_specs=pl.BlockSpec((bb, P, E), lambda b: (b, 0, 0)),
        out_shape=jax.ShapeDtypeStruct((B, P, E), x.dtype),
        compiler_params=pltpu.CompilerParams(
            dimension_semantics=("arbitrary",),
        ),
    )(idx, x, spatial_pos_embed, image_pos_embed)
